# baseline (speedup 1.0000x reference)
import functools

import jax
import jax.numpy as jnp
from jax import lax
from jax.experimental import pallas as pl
from jax.experimental.pallas import tpu as pltpu
from jax.experimental.pallas import tpu_sc as plsc

BATCH = 16384
COLS = 4
TOT = BATCH * COLS
NC, NS, L = 2, 16, 16
NW = NC * NS
ROWS_W = 512

_mesh = plsc.VectorSubcoreMesh(core_axis_name="c", subcore_axis_name="s")


@functools.partial(
    pl.kernel,
    mesh=_mesh,
    out_type=jax.ShapeDtypeStruct((TOT, 1), jnp.float32),
    scratch_types=[
        pltpu.VMEM((ROWS_W, 1), jnp.float32),
    ],
)
def _p0(embed, out, val_v):
    wid = lax.axis_index("s") * NC + lax.axis_index("c")
    for k in range(4):
        b = wid * (4 * ROWS_W) + k * ROWS_W
        pltpu.sync_copy(embed.at[pl.ds(b, ROWS_W), :], val_v)
        pltpu.sync_copy(val_v, out.at[pl.ds(b, ROWS_W), :])


def kernel(X, embed):
    return _p0(embed).reshape(BATCH, COLS)


# speedup vs baseline: 3.1351x; 3.1351x over previous
import functools

import jax
import jax.numpy as jnp
from jax import lax
from jax.experimental import pallas as pl
from jax.experimental.pallas import tpu as pltpu
from jax.experimental.pallas import tpu_sc as plsc

BATCH = 16384
COLS = 4
TOT = BATCH * COLS
NC, NS, L = 2, 16, 16
NW = NC * NS
ROWS_W = BATCH // NW

_mesh = plsc.VectorSubcoreMesh(core_axis_name="c", subcore_axis_name="s")


@functools.partial(
    pl.kernel,
    mesh=_mesh,
    out_type=jax.ShapeDtypeStruct((TOT,), jnp.float32),
    scratch_types=(
        [pltpu.VMEM((ROWS_W,), jnp.int32) for _ in range(COLS)]
        + [pltpu.VMEM((ROWS_W,), jnp.float32) for _ in range(COLS)]
        + [pltpu.VMEM((ROWS_W,), jnp.float32) for _ in range(COLS)]
        + [pltpu.SemaphoreType.DMA((COLS,)),
           pltpu.SemaphoreType.DMA((COLS,)),
           pltpu.SemaphoreType.DMA((COLS,))]
    ),
)
def _gather_sc(xt, embed, s_cm, *refs):
    idx_v = refs[0:COLS]
    val_v = refs[COLS:2 * COLS]
    out_v = refs[2 * COLS:3 * COLS]
    sem_i, sem_g, sem_o = refs[3 * COLS:]

    wid = lax.axis_index("s") * NC + lax.axis_index("c")
    rbase = wid * ROWS_W

    idx_cp = [
        pltpu.async_copy(xt.at[pl.ds(c * BATCH + rbase, ROWS_W)], idx_v[c],
                         sem_i.at[c])
        for c in range(COLS)
    ]
    gathers = []
    for c in range(COLS):
        idx_cp[c].wait()
        gathers.append(
            pltpu.async_copy(embed.at[idx_v[c]], val_v[c], sem_g.at[c]))
    for g in gathers:
        g.wait()

    def body(i, carry):
        sl = pl.ds(i * L, L)
        v = [val_v[c][sl] for c in range(COLS)]
        t = (v[0] + v[1]) + (v[2] + v[3])
        for c in range(COLS):
            out_v[c][sl] = 4.0 * v[c] - t
        return carry

    lax.fori_loop(0, ROWS_W // L, body, 0)
    out_cp = [
        pltpu.async_copy(out_v[c], s_cm.at[pl.ds(c * BATCH + rbase, ROWS_W)],
                         sem_o.at[c])
        for c in range(COLS)
    ]
    for cp in out_cp:
        cp.wait()


def _transpose_tc(s_ref, o_ref):
    o_ref[...] = jnp.transpose(s_ref[...])


_transpose_call = pl.pallas_call(
    _transpose_tc,
    grid=(BATCH // 512,),
    in_specs=[pl.BlockSpec((COLS, 512), lambda i: (0, i))],
    out_specs=pl.BlockSpec((512, COLS), lambda i: (i, 0)),
    out_shape=jax.ShapeDtypeStruct((BATCH, COLS), jnp.float32),
)


def kernel(X, embed):
    xt = X.T.reshape(TOT)
    ef = embed.reshape(embed.shape[0])
    s_cm = _gather_sc(xt, ef)
    return _transpose_call(s_cm.reshape(COLS, BATCH))


# speedup vs baseline: 4.2743x; 1.3634x over previous
import functools

import jax
import jax.numpy as jnp
from jax import lax
from jax.experimental import pallas as pl
from jax.experimental.pallas import tpu as pltpu
from jax.experimental.pallas import tpu_sc as plsc

BATCH = 16384
COLS = 4
TOT = BATCH * COLS
NC, NS, L = 2, 16, 16
NW = NC * NS
ROWS_W = BATCH // NW

_mesh = plsc.VectorSubcoreMesh(core_axis_name="c", subcore_axis_name="s")


@functools.partial(
    pl.kernel,
    mesh=_mesh,
    out_type=jax.ShapeDtypeStruct((TOT,), jnp.float32),
    scratch_types=(
        [pltpu.VMEM((ROWS_W,), jnp.int32) for _ in range(COLS)]
        + [pltpu.VMEM((ROWS_W,), jnp.float32) for _ in range(COLS)]
        + [pltpu.VMEM((ROWS_W,), jnp.float32) for _ in range(COLS)]
        + [pltpu.SemaphoreType.DMA((COLS,)),
           pltpu.SemaphoreType.DMA((COLS,)),
           pltpu.SemaphoreType.DMA((COLS,))]
    ),
)
def _gather_sc(xt, embed, s_cm, *refs):
    idx_v = refs[0:COLS]
    val_v = refs[COLS:2 * COLS]
    out_v = refs[2 * COLS:3 * COLS]
    sem_i, sem_g, sem_o = refs[3 * COLS:]

    wid = lax.axis_index("s") * NC + lax.axis_index("c")
    rbase = wid * ROWS_W

    idx_cp = [
        pltpu.async_copy(xt.at[pl.ds(c * BATCH + rbase, ROWS_W)], idx_v[c],
                         sem_i.at[c])
        for c in range(COLS)
    ]
    gathers = []
    for c in range(COLS):
        idx_cp[c].wait()
        gathers.append(
            pltpu.async_copy(embed.at[idx_v[c]], val_v[c], sem_g.at[c]))
    for g in gathers:
        g.wait()

    def body(i, carry):
        sl = pl.ds(i * L, L)
        v = [val_v[c][sl] for c in range(COLS)]
        t = (v[0] + v[1]) + (v[2] + v[3])
        for c in range(COLS):
            out_v[c][sl] = 4.0 * v[c] - t
        return carry

    lax.fori_loop(0, ROWS_W // L, body, 0)
    out_cp = [
        pltpu.async_copy(out_v[c], s_cm.at[pl.ds(c * BATCH + rbase, ROWS_W)],
                         sem_o.at[c])
        for c in range(COLS)
    ]
    for cp in out_cp:
        cp.wait()


def _transpose_tc(s_ref, o_ref):
    o_ref[...] = jnp.transpose(s_ref[...])


_transpose_call = pl.pallas_call(
    _transpose_tc,
    grid=(BATCH // 512,),
    in_specs=[pl.BlockSpec((COLS, 512), lambda i: (0, i))],
    out_specs=pl.BlockSpec((512, COLS), lambda i: (i, 0)),
    out_shape=jax.ShapeDtypeStruct((BATCH, COLS), jnp.float32),
)


def kernel(X, embed):
    xt = X.T.reshape(TOT)
    ef = embed.reshape(embed.shape[0])
    s_cm = _gather_sc(xt, ef)
    return s_cm.reshape(COLS, BATCH).T


# speedup vs baseline: 4.2746x; 1.0001x over previous
import functools

import jax
import jax.numpy as jnp
from jax import lax
from jax.experimental import pallas as pl
from jax.experimental.pallas import tpu as pltpu
from jax.experimental.pallas import tpu_sc as plsc

BATCH = 16384
COLS = 4
TOT = BATCH * COLS
NC, NS, L = 2, 16, 16
NW = NC * NS
ROWS_W = BATCH // NW

_mesh = plsc.VectorSubcoreMesh(core_axis_name="c", subcore_axis_name="s")


@functools.partial(
    pl.kernel,
    mesh=_mesh,
    out_type=jax.ShapeDtypeStruct((TOT,), jnp.float32),
    scratch_types=(
        [pltpu.VMEM((ROWS_W,), jnp.int32) for _ in range(COLS)]
        + [pltpu.VMEM((ROWS_W,), jnp.float32) for _ in range(COLS)]
        + [pltpu.VMEM((ROWS_W,), jnp.float32) for _ in range(COLS)]
        + [pltpu.SemaphoreType.DMA((COLS,)),
           pltpu.SemaphoreType.DMA((COLS,)),
           pltpu.SemaphoreType.DMA((COLS,))]
    ),
)
def _gather_sc(xt, embed, s_cm, *refs):
    idx_v = refs[0:COLS]
    val_v = refs[COLS:2 * COLS]
    out_v = refs[2 * COLS:3 * COLS]
    sem_i, sem_g, sem_o = refs[3 * COLS:]

    wid = lax.axis_index("s") * NC + lax.axis_index("c")
    rbase = wid * ROWS_W

    idx_cp = [
        pltpu.async_copy(xt.at[pl.ds(c * BATCH + rbase, ROWS_W)], idx_v[c],
                         sem_i.at[c])
        for c in range(COLS)
    ]
    gathers = []
    for c in range(COLS):
        idx_cp[c].wait()
        gathers.append(
            pltpu.async_copy(embed.at[idx_v[c]], val_v[c], sem_g.at[c]))
    for g in gathers:
        g.wait()

    def body(i, carry):
        sl = pl.ds(i * L, L)
        v = [val_v[c][sl] for c in range(COLS)]
        t = (v[0] + v[1]) + (v[2] + v[3])
        for c in range(COLS):
            out_v[c][sl] = 4.0 * v[c] - t
        return carry

    lax.fori_loop(0, ROWS_W // L, body, 0)
    out_cp = [
        pltpu.async_copy(out_v[c], s_cm.at[pl.ds(c * BATCH + rbase, ROWS_W)],
                         sem_o.at[c])
        for c in range(COLS)
    ]
    for cp in out_cp:
        cp.wait()


def _transpose_tc(s_ref, o_ref):
    o_ref[...] = jnp.transpose(s_ref[...])


_transpose_call = pl.pallas_call(
    _transpose_tc,
    grid=(BATCH // 512,),
    in_specs=[pl.BlockSpec((COLS, 512), lambda i: (0, i))],
    out_specs=pl.BlockSpec((512, COLS), lambda i: (i, 0)),
    out_shape=jax.ShapeDtypeStruct((BATCH, COLS), jnp.float32),
)


def kernel(X, embed):
    xt = X.T.reshape(TOT)
    ef = lax.reshape(embed, (embed.shape[0],), dimensions=(1, 0))
    s_cm = _gather_sc(xt, ef)
    return s_cm.reshape(COLS, BATCH).T
